# baseline (speedup 1.0000x reference)
import functools
import math

import jax
import jax.numpy as jnp
import numpy as np
from jax import lax
from jax.experimental import pallas as pl
from jax.experimental.pallas import tpu as pltpu
from jax.experimental.pallas import tpu_sc as plsc

_EMBED = 32
_LANES = 16


def _make_sc_scatter(B, L, n_idx):
    NC, NS = 2, 16
    NW = NC * NS
    nb = B // NW
    ni = n_idx // NW
    n_vec = ni // _LANES

    mesh = plsc.VectorSubcoreMesh(core_axis_name="c", subcore_axis_name="s")

    @functools.partial(
        pl.kernel,
        mesh=mesh,
        out_type=jax.ShapeDtypeStruct((B, L), jnp.float32),
        scratch_types=[
            pltpu.VMEM((ni,), jnp.int32),
            pltpu.VMEM((ni,), jnp.int32),
            pltpu.VMEM((nb, L), jnp.float32),
        ],
        compiler_params=pltpu.CompilerParams(
            use_tc_tiling_on_sc=False, needs_layout_passes=False),
    )
    def sc_scatter(rows_hbm, cols_hbm, out_hbm, rowv, colv, delta_v):
        wid = lax.axis_index("s") * NC + lax.axis_index("c")
        pltpu.sync_copy(rows_hbm.at[wid], rowv)
        pltpu.sync_copy(cols_hbm.at[wid], colv)

        zeros16 = jnp.zeros((_LANES,), jnp.float32)
        ones16 = jnp.ones((_LANES,), jnp.float32)

        def zero_body(i, carry):
            r = i // (L // _LANES)
            c = (i % (L // _LANES)) * _LANES
            delta_v[r, pl.ds(c, _LANES)] = zeros16
            return carry

        lax.fori_loop(0, nb * (L // _LANES), zero_body, 0)

        def scat_body(i, carry):
            rv = rowv[pl.ds(i * _LANES, _LANES)]
            cv = colv[pl.ds(i * _LANES, _LANES)]
            plsc.store_scatter(delta_v, [rv, cv], ones16)
            return carry

        lax.fori_loop(0, n_vec, scat_body, 0)

        iot = lax.iota(jnp.int32, _LANES)
        zcol = jnp.zeros((_LANES,), jnp.int32)
        for base in range(0, nb, _LANES):
            plsc.store_scatter(delta_v, [iot + base, zcol], zeros16)

        pltpu.sync_copy(delta_v, out_hbm.at[pl.ds(wid * nb, nb)])

    return sc_scatter


def _tc_body(TB, F, s_ref, d_ref, v_ref, c_ref, o_ref):
    s = s_ref[...]
    dl = d_ref[...]
    v = v_ref[...]
    W = c_ref[0:1, :].reshape(1, 1, F)
    D = c_ref[1:2, :].reshape(1, 1, F)
    C = c_ref[2:3, :].reshape(1, 1, F)
    S = c_ref[3:4, :].reshape(1, 1, F)
    K = c_ref[4:5, :].reshape(1, 1, F)
    A = c_ref[5:6, :].reshape(1, 1, F)
    phase = A * v.reshape(TB, 1, 1)
    emb = C + S * jnp.sin(phase) + K * jnp.cos(phase)
    o_ref[...] = s[:, :, None] * W + dl[:, :, None] * D + emb


def _make_tc_assemble(B, L, F, TB):
    grid = (B // TB,)
    return pl.pallas_call(
        functools.partial(_tc_body, TB, F),
        grid=grid,
        in_specs=[
            pl.BlockSpec((TB, L), lambda i: (i, 0)),
            pl.BlockSpec((TB, L), lambda i: (i, 0)),
            pl.BlockSpec((TB, 1), lambda i: (i, 0)),
            pl.BlockSpec((8, F), lambda i: (0, 0)),
        ],
        out_specs=pl.BlockSpec((TB, L, F), lambda i: (i, 0, 0)),
        out_shape=jax.ShapeDtypeStruct((B, L, F), jnp.float32),
    )


def kernel(series, peaks, variance, conv_kernel, conv_bias):
    B, L, C = series.shape
    Bp, P, _ = peaks.shape
    feat = conv_kernel.shape[1]
    F = 2 * _EMBED
    NW = 32
    nb = B // NW

    cols = jnp.clip(peaks.reshape(B, P), 0, L - 1).astype(jnp.int32)
    rows_local = (np.arange(B, dtype=np.int32) % nb).repeat(P)
    ni = (B * P) // NW
    rows_hbm = jnp.asarray(rows_local).reshape(NW, ni)
    cols_hbm = cols.reshape(NW, ni)

    delta = _make_sc_scatter(B, L, B * P)(rows_hbm, cols_hbm)

    zpad = jnp.zeros((F - feat,), jnp.float32)
    W64 = jnp.concatenate([conv_kernel[0].astype(jnp.float32), zpad])
    C64 = jnp.concatenate([conv_bias.astype(jnp.float32), zpad])
    D64 = jnp.asarray(np.eye(1, F, feat, dtype=np.float32)[0])
    half = _EMBED // 2
    S64 = np.zeros((F,), np.float32)
    S64[_EMBED:_EMBED + half] = 1.0
    K64 = np.zeros((F,), np.float32)
    K64[_EMBED + half:] = 1.0
    ang = 2.0 * math.pi * np.exp(
        np.linspace(0.0, math.log(1000.0), half)).astype(np.float32)
    A64 = np.zeros((F,), np.float32)
    A64[_EMBED:_EMBED + half] = ang
    A64[_EMBED + half:] = ang
    consts = jnp.stack([
        W64, D64, C64,
        jnp.asarray(S64), jnp.asarray(K64), jnp.asarray(A64),
        jnp.zeros((F,), jnp.float32), jnp.zeros((F,), jnp.float32),
    ])

    series2d = series.reshape(B, L)
    var2d = variance.reshape(B, 1)

    TB = 8
    out = _make_tc_assemble(B, L, F, TB)(series2d, delta, var2d, consts)
    return out


# speedup vs baseline: 1.1445x; 1.1445x over previous
import functools
import math

import jax
import jax.numpy as jnp
import numpy as np
from jax import lax
from jax.experimental import pallas as pl
from jax.experimental.pallas import tpu as pltpu
from jax.experimental.pallas import tpu_sc as plsc

_EMBED = 32
_LANES = 16


def _make_sc_scatter(B, L, n_idx):
    NC, NS = 2, 16
    NW = NC * NS
    nb = B // NW
    ni = n_idx // NW
    n_vec = ni // _LANES

    mesh = plsc.VectorSubcoreMesh(core_axis_name="c", subcore_axis_name="s")

    @functools.partial(
        pl.kernel,
        mesh=mesh,
        out_type=jax.ShapeDtypeStruct((B, L), jnp.int32),
        scratch_types=[
            pltpu.VMEM((ni,), jnp.int32),
            pltpu.VMEM((ni,), jnp.int32),
            pltpu.VMEM((nb, L), jnp.int32),
        ],
        compiler_params=pltpu.CompilerParams(
            use_tc_tiling_on_sc=False, needs_layout_passes=False),
    )
    def sc_scatter(rows_hbm, cols_hbm, out_hbm, rowv, colv, delta_v):
        wid = lax.axis_index("s") * NC + lax.axis_index("c")
        pltpu.sync_copy(rows_hbm.at[wid], rowv)
        pltpu.sync_copy(cols_hbm.at[wid], colv)

        zeros16 = jnp.zeros((_LANES,), jnp.int32)
        ones16 = jnp.ones((_LANES,), jnp.int32)

        def zero_body(i, carry):
            r = i // (L // _LANES)
            c = (i % (L // _LANES)) * _LANES
            delta_v[r, pl.ds(c, _LANES)] = zeros16
            return carry

        lax.fori_loop(0, nb * (L // _LANES), zero_body, 0)

        def scat_body(i, carry):
            rv = rowv[pl.ds(i * _LANES, _LANES)]
            cv = colv[pl.ds(i * _LANES, _LANES)]
            plsc.store_scatter(delta_v, [rv, cv], ones16)
            return carry

        lax.fori_loop(0, n_vec, scat_body, 0)

        iot = lax.iota(jnp.int32, _LANES)
        zcol = jnp.zeros((_LANES,), jnp.int32)
        for base in range(0, nb, _LANES):
            plsc.store_scatter(delta_v, [iot + base, zcol], zeros16)

        pltpu.sync_copy(delta_v, out_hbm.at[pl.ds(wid * nb, nb)])

    return sc_scatter


def _tc_body(TB, F, s_ref, d_ref, v_ref, c_ref, o_ref):
    s = s_ref[...]
    di = d_ref[...]
    v = v_ref[...]
    W = c_ref[0:1, :].reshape(1, 1, F)
    C = c_ref[2:3, :].reshape(1, 1, F)
    S = c_ref[3:4, :].reshape(1, 1, F)
    K = c_ref[4:5, :].reshape(1, 1, F)
    A = c_ref[5:6, :].reshape(1, 1, F)
    phase = A * v.reshape(TB, 1, 1)
    emb = C + S * jnp.sin(phase) + K * jnp.cos(phase)
    u = (lax.bitcast_convert_type(s, jnp.int32) & jnp.int32(~1)) | di
    ub = u[:, :, None] | jnp.zeros((1, 1, F), jnp.int32)
    sv = lax.bitcast_convert_type(ub, jnp.float32) * W + emb
    db = (ub & jnp.int32(1)).astype(jnp.float32)
    lane = lax.broadcasted_iota(jnp.int32, (1, 1, F), 2)
    o_ref[...] = jnp.where(lane == 31, db, sv)


def _make_tc_assemble(B, L, F, TB):
    grid = (B // TB,)
    return pl.pallas_call(
        functools.partial(_tc_body, TB, F),
        grid=grid,
        in_specs=[
            pl.BlockSpec((TB, L), lambda i: (i, 0)),
            pl.BlockSpec((TB, L), lambda i: (i, 0)),
            pl.BlockSpec((TB, 1), lambda i: (i, 0)),
            pl.BlockSpec((8, F), lambda i: (0, 0)),
        ],
        out_specs=pl.BlockSpec((TB, L, F), lambda i: (i, 0, 0)),
        out_shape=jax.ShapeDtypeStruct((B, L, F), jnp.float32),
    )


def kernel(series, peaks, variance, conv_kernel, conv_bias):
    B, L, C = series.shape
    Bp, P, _ = peaks.shape
    feat = conv_kernel.shape[1]
    F = 2 * _EMBED
    NW = 32
    nb = B // NW

    cols = jnp.clip(peaks.reshape(B, P), 0, L - 1).astype(jnp.int32)
    rows_local = (np.arange(B, dtype=np.int32) % nb).repeat(P)
    ni = (B * P) // NW
    rows_hbm = jnp.asarray(rows_local).reshape(NW, ni)
    cols_hbm = cols.reshape(NW, ni)

    delta = _make_sc_scatter(B, L, B * P)(rows_hbm, cols_hbm)

    zpad = jnp.zeros((F - feat,), jnp.float32)
    W64 = jnp.concatenate([conv_kernel[0].astype(jnp.float32), zpad])
    C64 = jnp.concatenate([conv_bias.astype(jnp.float32), zpad])
    D64 = jnp.asarray(np.eye(1, F, feat, dtype=np.float32)[0])
    half = _EMBED // 2
    S64 = np.zeros((F,), np.float32)
    S64[_EMBED:_EMBED + half] = 1.0
    K64 = np.zeros((F,), np.float32)
    K64[_EMBED + half:] = 1.0
    ang = 2.0 * math.pi * np.exp(
        np.linspace(0.0, math.log(1000.0), half)).astype(np.float32)
    A64 = np.zeros((F,), np.float32)
    A64[_EMBED:_EMBED + half] = ang
    A64[_EMBED + half:] = ang
    consts = jnp.stack([
        W64, D64, C64,
        jnp.asarray(S64), jnp.asarray(K64), jnp.asarray(A64),
        jnp.zeros((F,), jnp.float32), jnp.zeros((F,), jnp.float32),
    ])

    series2d = series.reshape(B, L)
    var2d = variance.reshape(B, 1)

    TB = 8
    out = _make_tc_assemble(B, L, F, TB)(series2d, delta, var2d, consts)
    return out


# speedup vs baseline: 1.1495x; 1.0044x over previous
import functools
import math

import jax
import jax.numpy as jnp
import numpy as np
from jax import lax
from jax.experimental import pallas as pl
from jax.experimental.pallas import tpu as pltpu
from jax.experimental.pallas import tpu_sc as plsc

_EMBED = 32
_LANES = 16


def _make_sc_scatter(B, L, n_idx):
    NC, NS = 2, 16
    NW = NC * NS
    nb = B // NW
    ni = n_idx // NW
    n_vec = ni // _LANES

    mesh = plsc.VectorSubcoreMesh(core_axis_name="c", subcore_axis_name="s")

    @functools.partial(
        pl.kernel,
        mesh=mesh,
        out_type=jax.ShapeDtypeStruct((B, L), jnp.int32),
        scratch_types=[
            pltpu.VMEM((ni,), jnp.int32),
            pltpu.VMEM((ni,), jnp.int32),
            pltpu.VMEM((nb, L), jnp.int32),
        ],
        compiler_params=pltpu.CompilerParams(
            use_tc_tiling_on_sc=False, needs_layout_passes=False),
    )
    def sc_scatter(rows_hbm, cols_hbm, out_hbm, rowv, colv, delta_v):
        wid = lax.axis_index("s") * NC + lax.axis_index("c")
        pltpu.sync_copy(rows_hbm.at[wid], rowv)
        pltpu.sync_copy(cols_hbm.at[wid], colv)

        zeros16 = jnp.zeros((_LANES,), jnp.int32)
        ones16 = jnp.ones((_LANES,), jnp.int32)

        def zero_body(i, carry):
            r = i // (L // _LANES)
            c = (i % (L // _LANES)) * _LANES
            delta_v[r, pl.ds(c, _LANES)] = zeros16
            return carry

        lax.fori_loop(0, nb * (L // _LANES), zero_body, 0)

        def scat_body(i, carry):
            rv = rowv[pl.ds(i * _LANES, _LANES)]
            cv = colv[pl.ds(i * _LANES, _LANES)]
            plsc.store_scatter(delta_v, [rv, cv], ones16)
            return carry

        lax.fori_loop(0, n_vec, scat_body, 0)

        iot = lax.iota(jnp.int32, _LANES)
        zcol = jnp.zeros((_LANES,), jnp.int32)
        for base in range(0, nb, _LANES):
            plsc.store_scatter(delta_v, [iot + base, zcol], zeros16)

        pltpu.sync_copy(delta_v, out_hbm.at[pl.ds(wid * nb, nb)])

    return sc_scatter


_NSPLIT = 4


def _tc_body(TB, F, nsteps, s_ref, d_ref, v_ref, c_ref, o_any, buf, sem):
    i = pl.program_id(0)
    slot = lax.rem(i, 2)
    sub = TB // _NSPLIT

    @pl.when(i >= 2)
    def _():
        for k in range(_NSPLIT):
            pltpu.make_async_copy(
                buf.at[slot, pl.ds(k * sub, sub)],
                o_any.at[pl.ds((i - 2) * TB + k * sub, sub)],
                sem.at[slot, k]).wait()

    s = s_ref[...]
    di = d_ref[...]
    v = v_ref[...]
    W = c_ref[0:1, :].reshape(1, 1, F)
    C = c_ref[2:3, :].reshape(1, 1, F)
    S = c_ref[3:4, :].reshape(1, 1, F)
    K = c_ref[4:5, :].reshape(1, 1, F)
    A = c_ref[5:6, :].reshape(1, 1, F)
    phase = A * v.reshape(TB, 1, 1)
    emb = C + S * jnp.sin(phase) + K * jnp.cos(phase)
    u = (lax.bitcast_convert_type(s, jnp.int32) & jnp.int32(~1)) | di
    ub = u[:, :, None] | jnp.zeros((1, 1, F), jnp.int32)
    sv = lax.bitcast_convert_type(ub, jnp.float32) * W + emb
    db = (ub & jnp.int32(1)).astype(jnp.float32)
    lane = lax.broadcasted_iota(jnp.int32, (1, 1, F), 2)
    buf[slot] = jnp.where(lane == 31, db, sv)

    for k in range(_NSPLIT):
        pltpu.make_async_copy(
            buf.at[slot, pl.ds(k * sub, sub)],
            o_any.at[pl.ds(i * TB + k * sub, sub)],
            sem.at[slot, k]).start()

    @pl.when(i == nsteps - 1)
    def _():
        for k in range(_NSPLIT):
            pltpu.make_async_copy(
                buf.at[slot, pl.ds(k * sub, sub)],
                o_any.at[pl.ds(i * TB + k * sub, sub)],
                sem.at[slot, k]).wait()
        other = lax.rem(i + 1, 2)

        @pl.when(i >= 1)
        def _():
            for k in range(_NSPLIT):
                pltpu.make_async_copy(
                    buf.at[other, pl.ds(k * sub, sub)],
                    o_any.at[pl.ds((i - 1) * TB + k * sub, sub)],
                    sem.at[other, k]).wait()


def _make_tc_assemble(B, L, F, TB):
    nsteps = B // TB
    return pl.pallas_call(
        functools.partial(_tc_body, TB, F, nsteps),
        grid=(nsteps,),
        in_specs=[
            pl.BlockSpec((TB, L), lambda i: (i, 0)),
            pl.BlockSpec((TB, L), lambda i: (i, 0)),
            pl.BlockSpec((TB, 1), lambda i: (i, 0)),
            pl.BlockSpec((8, F), lambda i: (0, 0)),
        ],
        out_specs=pl.BlockSpec(memory_space=pl.ANY),
        out_shape=jax.ShapeDtypeStruct((B, L, F), jnp.float32),
        scratch_shapes=[
            pltpu.VMEM((2, TB, L, F), jnp.float32),
            pltpu.SemaphoreType.DMA((2, _NSPLIT)),
        ],
    )


def kernel(series, peaks, variance, conv_kernel, conv_bias):
    B, L, C = series.shape
    Bp, P, _ = peaks.shape
    feat = conv_kernel.shape[1]
    F = 2 * _EMBED
    NW = 32
    nb = B // NW

    cols = jnp.clip(peaks.reshape(B, P), 0, L - 1).astype(jnp.int32)
    rows_local = (np.arange(B, dtype=np.int32) % nb).repeat(P)
    ni = (B * P) // NW
    rows_hbm = jnp.asarray(rows_local).reshape(NW, ni)
    cols_hbm = cols.reshape(NW, ni)

    delta = _make_sc_scatter(B, L, B * P)(rows_hbm, cols_hbm)

    zpad = jnp.zeros((F - feat,), jnp.float32)
    W64 = jnp.concatenate([conv_kernel[0].astype(jnp.float32), zpad])
    C64 = jnp.concatenate([conv_bias.astype(jnp.float32), zpad])
    D64 = jnp.asarray(np.eye(1, F, feat, dtype=np.float32)[0])
    half = _EMBED // 2
    S64 = np.zeros((F,), np.float32)
    S64[_EMBED:_EMBED + half] = 1.0
    K64 = np.zeros((F,), np.float32)
    K64[_EMBED + half:] = 1.0
    ang = 2.0 * math.pi * np.exp(
        np.linspace(0.0, math.log(1000.0), half)).astype(np.float32)
    A64 = np.zeros((F,), np.float32)
    A64[_EMBED:_EMBED + half] = ang
    A64[_EMBED + half:] = ang
    consts = jnp.stack([
        W64, D64, C64,
        jnp.asarray(S64), jnp.asarray(K64), jnp.asarray(A64),
        jnp.zeros((F,), jnp.float32), jnp.zeros((F,), jnp.float32),
    ])

    series2d = series.reshape(B, L)
    var2d = variance.reshape(B, 1)

    TB = 8
    out = _make_tc_assemble(B, L, F, TB)(series2d, delta, var2d, consts)
    return out
